# Initial kernel scaffold; baseline (speedup 1.0000x reference)
import jax
import jax.numpy as jnp
from jax.experimental import pallas as pl


def kernel(input_ids, table):
    raise NotImplementedError("write your pallas kernel here")


# speedup vs baseline: 1.4688x; 1.4688x over previous
import functools

import jax
import jax.numpy as jnp
from jax import lax
from jax.experimental import pallas as pl
from jax.experimental.pallas import tpu as pltpu
from jax.experimental.pallas import tpu_sc as plsc

DIM = 32

_info = plsc.get_sparse_core_info()
_NC, _NS = _info.num_cores, _info.num_subcores
_NW = _NC * _NS


@functools.partial(jax.jit, static_argnums=(2, 3))
def _sc_gather(ids_flat, table, chunk, n_chunks):
    per_w = chunk * n_chunks
    b_total = per_w * _NW
    mesh = plsc.VectorSubcoreMesh(core_axis_name="c", subcore_axis_name="s")

    @functools.partial(
        pl.kernel,
        out_type=jax.ShapeDtypeStruct((b_total, DIM), jnp.float32),
        mesh=mesh,
        scratch_types=[
            pltpu.VMEM((chunk,), jnp.int32),
            pltpu.VMEM((chunk, DIM), jnp.float32),
            pltpu.SemaphoreType.DMA,
        ],
        compiler_params=pltpu.CompilerParams(use_tc_tiling_on_sc=False),
    )
    def k(ids_hbm, table_hbm, out_hbm, idx_v, rows_v, sem):
        wid = lax.axis_index("s") * _NC + lax.axis_index("c")
        base = wid * per_w

        def body(g, carry):
            off = base + g * chunk
            pltpu.sync_copy(ids_hbm.at[pl.ds(off, chunk)], idx_v)
            pltpu.async_copy(table_hbm.at[idx_v], rows_v, sem).wait()
            pltpu.sync_copy(rows_v, out_hbm.at[pl.ds(off, chunk)])
            return carry

        lax.fori_loop(0, n_chunks, body, 0)

    return k(ids_flat, table)


def kernel(input_ids, table):
    bsz, seq = input_ids.shape
    ids_flat = input_ids.reshape(-1).astype(jnp.int32)
    per_w = (bsz * seq) // _NW
    chunk = 1280
    n_chunks = per_w // chunk
    out = _sc_gather(ids_flat, table, chunk, n_chunks)
    return out.reshape(bsz, seq, DIM)


# speedup vs baseline: 1.4943x; 1.0173x over previous
import functools

import jax
import jax.numpy as jnp
from jax import lax
from jax.experimental import pallas as pl
from jax.experimental.pallas import tpu as pltpu
from jax.experimental.pallas import tpu_sc as plsc

DIM = 32

_info = plsc.get_sparse_core_info()
_NC, _NS = _info.num_cores, _info.num_subcores
_NW = _NC * _NS


@functools.partial(jax.jit, static_argnums=(2, 3))
def _sc_gather(ids_flat, table, chunk, n_chunks):
    per_w = chunk * n_chunks
    b_total = per_w * _NW
    mesh = plsc.VectorSubcoreMesh(core_axis_name="c", subcore_axis_name="s")

    @functools.partial(
        pl.kernel,
        out_type=jax.ShapeDtypeStruct((b_total, DIM), jnp.float32),
        mesh=mesh,
        scratch_types=[
            pltpu.VMEM((per_w,), jnp.int32),
            pltpu.VMEM((chunk, DIM), jnp.float32),
            pltpu.VMEM((chunk, DIM), jnp.float32),
            pltpu.SemaphoreType.DMA,
            pltpu.SemaphoreType.DMA,
        ],
        compiler_params=pltpu.CompilerParams(use_tc_tiling_on_sc=False),
    )
    def k(ids_hbm, table_hbm, out_hbm, idx_v, buf0, buf1, sem0, sem1):
        wid = lax.axis_index("s") * _NC + lax.axis_index("c")
        base = wid * per_w
        pltpu.sync_copy(ids_hbm.at[pl.ds(base, per_w)], idx_v)

        def gather(g, buf, sem):
            pltpu.async_copy(
                table_hbm.at[idx_v.at[pl.ds(g * chunk, chunk)]], buf, sem)

        def gather_wait(buf, sem):
            pltpu.make_async_copy(out_hbm.at[pl.ds(0, chunk)], buf, sem).wait()

        def store(g, buf):
            pltpu.sync_copy(buf, out_hbm.at[pl.ds(base + g * chunk, chunk)])

        gather(0, buf0, sem0)

        def body(i, carry):
            g = 2 * i
            gather_wait(buf0, sem0)
            gather(g + 1, buf1, sem1)
            store(g, buf0)
            gather_wait(buf1, sem1)

            @pl.when(g + 2 < n_chunks)
            def _():
                gather(g + 2, buf0, sem0)

            store(g + 1, buf1)
            return carry

        lax.fori_loop(0, n_chunks // 2, body, 0)

    return k(ids_flat, table)


def kernel(input_ids, table):
    bsz, seq = input_ids.shape
    ids_flat = input_ids.reshape(-1).astype(jnp.int32)
    per_w = (bsz * seq) // _NW
    chunk = 1280
    n_chunks = per_w // chunk
    out = _sc_gather(ids_flat, table, chunk, n_chunks)
    return out.reshape(bsz, seq, DIM)


# speedup vs baseline: 1.5003x; 1.0041x over previous
import functools

import jax
import jax.numpy as jnp
from jax import lax
from jax.experimental import pallas as pl
from jax.experimental.pallas import tpu as pltpu
from jax.experimental.pallas import tpu_sc as plsc

DIM = 32

_info = plsc.get_sparse_core_info()
_NC, _NS = _info.num_cores, _info.num_subcores
_NW = _NC * _NS


@functools.partial(jax.jit, static_argnums=(2, 3, 4))
def _sc_gather(ids_flat, table, chunk, n_chunks, nbuf):
    per_w = chunk * n_chunks
    b_total = per_w * _NW
    mesh = plsc.VectorSubcoreMesh(core_axis_name="c", subcore_axis_name="s")

    @functools.partial(
        pl.kernel,
        out_type=jax.ShapeDtypeStruct((b_total, DIM), jnp.float32),
        mesh=mesh,
        scratch_types=(
            [pltpu.VMEM((per_w,), jnp.int32)]
            + [pltpu.VMEM((chunk, DIM), jnp.float32) for _ in range(nbuf)]
            + [pltpu.SemaphoreType.DMA for _ in range(2 * nbuf)]
        ),
        compiler_params=pltpu.CompilerParams(use_tc_tiling_on_sc=False),
    )
    def k(ids_hbm, table_hbm, out_hbm, idx_v, *rest):
        bufs = rest[:nbuf]
        gsems = rest[nbuf:2 * nbuf]
        ssems = rest[2 * nbuf:]
        wid = lax.axis_index("s") * _NC + lax.axis_index("c")
        base = wid * per_w
        pltpu.sync_copy(ids_hbm.at[pl.ds(base, per_w)], idx_v)

        def gather(g, buf, sem):
            pltpu.async_copy(
                table_hbm.at[idx_v.at[pl.ds(g * chunk, chunk)]], buf, sem)

        def gather_wait(buf, sem):
            pltpu.make_async_copy(out_hbm.at[pl.ds(0, chunk)], buf, sem).wait()

        def store(g, buf, sem):
            pltpu.async_copy(buf, out_hbm.at[pl.ds(base + g * chunk, chunk)], sem)

        def store_wait(buf, sem):
            pltpu.make_async_copy(buf, out_hbm.at[pl.ds(0, chunk)], sem).wait()

        for b in range(nbuf):
            gather(b, bufs[b], gsems[b])

        def body(i, carry):
            g0 = i * nbuf
            for b in range(nbuf):
                g = g0 + b
                gather_wait(bufs[b], gsems[b])
                store(g, bufs[b], ssems[b])

                @pl.when(g + nbuf < n_chunks)
                def _():
                    store_wait(bufs[b], ssems[b])
                    gather(g + nbuf, bufs[b], gsems[b])

            return carry

        lax.fori_loop(0, n_chunks // nbuf, body, 0)
        for b in range(nbuf):
            store_wait(bufs[b], ssems[b])

    return k(ids_flat, table)


def kernel(input_ids, table):
    bsz, seq = input_ids.shape
    ids_flat = input_ids.reshape(-1).astype(jnp.int32)
    per_w = (bsz * seq) // _NW
    chunk = 640
    nbuf = 4
    n_chunks = per_w // chunk
    out = _sc_gather(ids_flat, table, chunk, n_chunks, nbuf)
    return out.reshape(bsz, seq, DIM)
